# Initial kernel scaffold; baseline (speedup 1.0000x reference)
import functools

import jax
import jax.numpy as jnp
from jax import lax
from jax.experimental import pallas as pl
from jax.experimental.pallas import tpu as pltpu
from jax.experimental.pallas import tpu_sc as plsc

N = 10000
E = 320000
D = 128

NC = 2
NS = 16
NW = NC * NS

CW = 125
EPW = E // NW
CPW = EPW // CW
ROWS = E // CW
RPS = N // NS

_MESH = plsc.VectorSubcoreMesh(core_axis_name="c", subcore_axis_name="s")

_f32 = jnp.float32


@functools.partial(
    pl.kernel,
    out_type=(
        jax.ShapeDtypeStruct((NC, N, 16), _f32),
        jax.ShapeDtypeStruct((NC, N, 16), _f32),
    ),
    mesh=_MESH,
    scratch_types=[
        pltpu.VMEM((CPW, CW), jnp.int32),
        pltpu.VMEM((CPW, CW), jnp.int32),
        pltpu.VMEM((CW, 16), _f32),
        pltpu.VMEM((RPS, 16), _f32),
        pltpu.VMEM_SHARED((N, 16), _f32),
        pltpu.VMEM_SHARED((N, 16), _f32),
    ],
)
def _sc_deg(src_hbm, dst_hbm, ones_hbm, zero_hbm,
            dop_hbm, dip_hbm,
            src_idx, dst_idx, ones_v, zb, acc_o, acc_i):
    c = lax.axis_index("c")
    s = lax.axis_index("s")
    w = s * NC + c
    pltpu.sync_copy(ones_hbm, ones_v)
    pltpu.sync_copy(zero_hbm, zb)
    pltpu.sync_copy(zb, acc_o.at[pl.ds(s * RPS, RPS)])
    pltpu.sync_copy(zb, acc_i.at[pl.ds(s * RPS, RPS)])
    pltpu.sync_copy(src_hbm.at[pl.ds(w * CPW, CPW)], src_idx)
    pltpu.sync_copy(dst_hbm.at[pl.ds(w * CPW, CPW)], dst_idx)
    plsc.subcore_barrier()
    for j in range(CPW):
        pltpu.sync_copy(ones_v, acc_o.at[src_idx.at[j]], add=True)
        pltpu.sync_copy(ones_v, acc_i.at[dst_idx.at[j]], add=True)
    plsc.subcore_barrier()
    pltpu.sync_copy(acc_o.at[pl.ds(s * RPS, RPS)], zb)
    pltpu.sync_copy(zb, dop_hbm.at[c, pl.ds(s * RPS, RPS)])
    pltpu.sync_copy(acc_i.at[pl.ds(s * RPS, RPS)], zb)
    pltpu.sync_copy(zb, dip_hbm.at[c, pl.ds(s * RPS, RPS)])


@functools.partial(
    pl.kernel,
    out_type=jax.ShapeDtypeStruct((NC, N, D), _f32),
    mesh=_MESH,
    scratch_types=[
        pltpu.VMEM((CPW, CW), jnp.int32),
        pltpu.VMEM((CPW, CW), jnp.int32),
        pltpu.VMEM((CW, D), _f32),
        pltpu.VMEM((CW, D), _f32),
        pltpu.VMEM((CW, D), _f32),
        pltpu.VMEM_SHARED((N, D), _f32),
        pltpu.SemaphoreType.DMA,
        pltpu.SemaphoreType.DMA,
    ],
)
def _sc_feat(h_hbm, src_hbm, dst_hbm, zero_hbm,
             out_hbm,
             src_idx, dst_idx, buf0, buf1, zb, agg_sh, sem0, sem1):
    c = lax.axis_index("c")
    s = lax.axis_index("s")
    w = s * NC + c
    pltpu.sync_copy(zero_hbm, zb)
    for t in range(RPS // CW):
        pltpu.sync_copy(zb, agg_sh.at[pl.ds(s * RPS + t * CW, CW)])
    pltpu.sync_copy(src_hbm.at[pl.ds(w * CPW, CPW)], src_idx)
    pltpu.sync_copy(dst_hbm.at[pl.ds(w * CPW, CPW)], dst_idx)
    plsc.subcore_barrier()
    bufs = (buf0, buf1)
    sems = (sem0, sem1)
    gh = [
        pltpu.async_copy(h_hbm.at[src_idx.at[0]], buf0, sem0),
        pltpu.async_copy(h_hbm.at[src_idx.at[1]], buf1, sem1),
    ]
    for j in range(CPW):
        b = j & 1
        gh[b].wait()
        pltpu.sync_copy(bufs[b], agg_sh.at[dst_idx.at[j]], add=True)
        if j + 2 < CPW:
            gh[b] = pltpu.async_copy(
                h_hbm.at[src_idx.at[j + 2]], bufs[b], sems[b])
    plsc.subcore_barrier()
    for t in range(RPS // CW):
        pltpu.sync_copy(agg_sh.at[pl.ds(s * RPS + t * CW, CW)], zb)
        pltpu.sync_copy(zb, out_hbm.at[c, pl.ds(s * RPS + t * CW, CW)])


_R = 2000


def _deg_r(p_ref):
    deg = p_ref[0, :, 0] + p_ref[1, :, 0]
    return lax.rsqrt(jnp.maximum(deg, 1.0))


def _prep_body(x_ref, dop_ref, o_ref):
    o_ref[...] = x_ref[...] * _deg_r(dop_ref)[:, None]


def _layer_body(aggp_ref, dip_ref, dop_ref, w_ref, b_ref, o_ref):
    rin = _deg_r(dip_ref)
    agg = (aggp_ref[0] + aggp_ref[1]) * rin[:, None]
    z = jnp.dot(agg, w_ref[...], preferred_element_type=_f32) + b_ref[...]
    h = jnp.maximum(z, 0.0)
    nrm = jnp.sqrt(jnp.sum(h * h, axis=1, keepdims=True))
    h = h / jnp.maximum(nrm, 1e-12)
    o_ref[...] = h * _deg_r(dop_ref)[:, None]


def _final_body(aggp_ref, dip_ref, x_ref, w_ref, b_ref, o_ref):
    rin = _deg_r(dip_ref)
    agg = (aggp_ref[0] + aggp_ref[1]) * rin[:, None]
    z = jnp.dot(agg, w_ref[...], preferred_element_type=_f32) + b_ref[...]
    h = jnp.maximum(z, 0.0)
    nrm = jnp.sqrt(jnp.sum(h * h, axis=1, keepdims=True))
    h = h / jnp.maximum(nrm, 1e-12)
    y = jnp.maximum(h + x_ref[...], 0.0)
    nrm2 = jnp.sqrt(jnp.sum(y * y, axis=1, keepdims=True))
    o_ref[...] = y / jnp.maximum(nrm2, 1e-12)


_spec_rows = pl.BlockSpec((_R, D), lambda i: (i, 0))
_spec_part = pl.BlockSpec((2, _R, 16), lambda i: (0, i, 0))
_spec_aggp = pl.BlockSpec((2, _R, D), lambda i: (0, i, 0))
_spec_w = pl.BlockSpec((D, D), lambda i: (0, 0))
_spec_b = pl.BlockSpec((1, D), lambda i: (0, 0))
_out_rows = jax.ShapeDtypeStruct((N, D), _f32)

_tc_prep = pl.pallas_call(
    _prep_body,
    grid=(N // _R,),
    in_specs=[_spec_rows, _spec_part],
    out_specs=_spec_rows,
    out_shape=_out_rows,
)

_tc_layer = pl.pallas_call(
    _layer_body,
    grid=(N // _R,),
    in_specs=[_spec_aggp, _spec_part, _spec_part, _spec_w, _spec_b],
    out_specs=_spec_rows,
    out_shape=_out_rows,
)

_tc_final = pl.pallas_call(
    _final_body,
    grid=(N // _R,),
    in_specs=[_spec_aggp, _spec_part, _spec_rows, _spec_w, _spec_b],
    out_specs=_spec_rows,
    out_shape=_out_rows,
)


def kernel(x, edge_index, W1, b1, W2, b2):
    src2d = edge_index[0].reshape(ROWS, CW)
    dst2d = edge_index[1].reshape(ROWS, CW)
    ones_blk = jnp.ones((CW, 16), _f32)
    zero16 = jnp.zeros((RPS, 16), _f32)
    zero_d = jnp.zeros((CW, D), _f32)

    dop, dip = _sc_deg(src2d, dst2d, ones_blk, zero16)
    xp = _tc_prep(x, dop)
    aggp1 = _sc_feat(xp, src2d, dst2d, zero_d)
    h1s = _tc_layer(aggp1, dip, dop, W1, b1.reshape(1, D))
    aggp2 = _sc_feat(h1s, src2d, dst2d, zero_d)
    out = _tc_final(aggp2, dip, x, W2, b2.reshape(1, D))
    return out


# speedup vs baseline: 1.7270x; 1.7270x over previous
import functools

import jax
import jax.numpy as jnp
from jax import lax
from jax.experimental import pallas as pl
from jax.experimental.pallas import tpu as pltpu
from jax.experimental.pallas import tpu_sc as plsc

N = 10000
E = 320000
D = 128

NC = 2
NS = 16

CW = 125
ROWS = E // CW
RPW = ROWS // NS
CPW = ROWS // (NC * NS)
NP = 10240
DS = NP // 8
DPS = DS // NS
QN = NP // 4
QP = QN + 8 * NS
SPS = QN // NS

_MESH = plsc.VectorSubcoreMesh(core_axis_name="c", subcore_axis_name="s")

_f32 = jnp.float32


@functools.partial(
    pl.kernel,
    out_type=(
        jax.ShapeDtypeStruct((NC * DS, D), _f32),
        jax.ShapeDtypeStruct((NC * DS, D), _f32),
    ),
    mesh=_MESH,
    scratch_types=[
        pltpu.VMEM((CPW, CW), jnp.int32),
        pltpu.VMEM((CPW, CW), jnp.int32),
        pltpu.VMEM((CW, D), _f32),
        pltpu.VMEM((DPS, D), _f32),
        pltpu.VMEM_SHARED((DS, D), _f32),
    ],
)
def _sc_deg(e8_hbm, i8s_hbm, ids_hbm, i8d_hbm, idd_hbm, zero_hbm,
            dop_hbm, dip_hbm,
            i8_v, id_v, buf, zb, acc):
    c = lax.axis_index("c")
    s = lax.axis_index("s")
    w = s * NC + c
    base = w * CPW
    for i8_hbm, id_hbm, o_hbm in ((i8s_hbm, ids_hbm, dop_hbm),
                                  (i8d_hbm, idd_hbm, dip_hbm)):
        pltpu.sync_copy(i8_hbm.at[pl.ds(base, CPW)], i8_v)
        pltpu.sync_copy(id_hbm.at[pl.ds(base, CPW)], id_v)
        pltpu.sync_copy(zero_hbm, zb)
        pltpu.sync_copy(zb, acc.at[pl.ds(s * DPS, DPS)])
        plsc.subcore_barrier()

        def body(j, _):
            pltpu.sync_copy(e8_hbm.at[i8_v.at[j]], buf)
            pltpu.sync_copy(buf, acc.at[id_v.at[j]], add=True)
            return 0

        lax.fori_loop(0, CPW, body, 0)
        plsc.subcore_barrier()
        pltpu.sync_copy(acc.at[pl.ds(s * DPS, DPS)], zb)
        pltpu.sync_copy(zb, o_hbm.at[pl.ds(c * DS + s * DPS, DPS)])


@functools.partial(
    pl.kernel,
    out_type=jax.ShapeDtypeStruct((4 * QN, D), _f32),
    mesh=_MESH,
    scratch_types=[
        pltpu.VMEM((RPW, CW), jnp.int32),
        pltpu.VMEM((RPW, CW), jnp.int32),
        pltpu.VMEM((CW, D), _f32),
        pltpu.VMEM((SPS, D), _f32),
        pltpu.VMEM_SHARED((QP, D), _f32),
    ],
)
def _sc_feat(h_hbm, src_hbm, dstl_hbm, zero_hbm,
             out_hbm,
             src_idx, dst_idx, buf0, zb, acc):
    c = lax.axis_index("c")
    s = lax.axis_index("s")
    base = s * RPW

    pltpu.sync_copy(src_hbm.at[pl.ds(base, RPW)], src_idx)
    pltpu.sync_copy(zero_hbm, zb)

    for p in range(2):
        q = 2 * p + c
        pltpu.sync_copy(dstl_hbm.at[pl.ds(q * ROWS + base, RPW)], dst_idx)
        pltpu.sync_copy(zb, acc.at[pl.ds(s * SPS, SPS)])
        pltpu.sync_copy(zb.at[pl.ds(0, 8)], acc.at[pl.ds(QN + 8 * s, 8)])
        plsc.subcore_barrier()

        def body(j, _):
            pltpu.sync_copy(h_hbm.at[src_idx.at[j]], buf0)
            pltpu.sync_copy(buf0, acc.at[dst_idx.at[j]], add=True)
            return 0

        lax.fori_loop(0, RPW, body, 0)

        plsc.subcore_barrier()
        pltpu.sync_copy(acc.at[pl.ds(s * SPS, SPS)], zb)
        pltpu.sync_copy(zb, out_hbm.at[pl.ds(q * QN + s * SPS, SPS)])
        pltpu.sync_copy(zero_hbm, zb)


_R = 2000


def _deg_r(p_ref):
    return lax.rsqrt(jnp.maximum(p_ref[:, 0], 1.0))


def _prep_body(x_ref, dop_ref, o_ref):
    o_ref[...] = x_ref[...] * _deg_r(dop_ref)[:, None]


def _gcn_post(aggp_ref, dip_ref, w_ref, b_ref):
    agg = aggp_ref[...] * _deg_r(dip_ref)[:, None]
    z = jnp.dot(agg, w_ref[...], preferred_element_type=_f32) + b_ref[...]
    h = jnp.maximum(z, 0.0)
    nrm = jnp.sqrt(jnp.sum(h * h, axis=1, keepdims=True))
    return h / jnp.maximum(nrm, 1e-12)


def _layer_body(aggp_ref, dip_ref, dop_ref, w_ref, b_ref, o_ref):
    h = _gcn_post(aggp_ref, dip_ref, w_ref, b_ref)
    o_ref[...] = h * _deg_r(dop_ref)[:, None]


def _final_body(aggp_ref, dip_ref, x_ref, w_ref, b_ref, o_ref):
    h = _gcn_post(aggp_ref, dip_ref, w_ref, b_ref)
    y = jnp.maximum(h + x_ref[...], 0.0)
    nrm = jnp.sqrt(jnp.sum(y * y, axis=1, keepdims=True))
    o_ref[...] = y / jnp.maximum(nrm, 1e-12)


_spec_rows = pl.BlockSpec((_R, D), lambda i: (i, 0))
_spec_deg = pl.BlockSpec((_R, 16), lambda i: (i, 0))
_spec_w = pl.BlockSpec((D, D), lambda i: (0, 0))
_spec_b = pl.BlockSpec((1, D), lambda i: (0, 0))
_out_rows = jax.ShapeDtypeStruct((N, D), _f32)

_tc_prep = pl.pallas_call(
    _prep_body,
    grid=(N // _R,),
    in_specs=[_spec_rows, _spec_deg],
    out_specs=_spec_rows,
    out_shape=_out_rows,
)

_tc_layer = pl.pallas_call(
    _layer_body,
    grid=(N // _R,),
    in_specs=[_spec_rows, _spec_deg, _spec_deg, _spec_w, _spec_b],
    out_specs=_spec_rows,
    out_shape=_out_rows,
)

_tc_final = pl.pallas_call(
    _final_body,
    grid=(N // _R,),
    in_specs=[_spec_rows, _spec_deg, _spec_rows, _spec_w, _spec_b],
    out_specs=_spec_rows,
    out_shape=_out_rows,
)


def kernel(x, edge_index, W1, b1, W2, b2):
    src = edge_index[0]
    dst = edge_index[1]
    src2d = src.reshape(ROWS, CW)
    e8 = jnp.zeros((8, D), _f32).at[jnp.arange(8), jnp.arange(8) * 16].set(1.0)
    i8s = (src % 8).reshape(ROWS, CW)
    ids_ = (src // 8).reshape(ROWS, CW)
    i8d = (dst % 8).reshape(ROWS, CW)
    idd = (dst // 8).reshape(ROWS, CW)
    dump = QN + 8 * (jnp.arange(E, dtype=jnp.int32) // (RPW * CW))
    loc = dst[None, :] - (jnp.arange(4, dtype=jnp.int32) * QN)[:, None]
    dstl = jnp.where((loc >= 0) & (loc < QN), loc, dump[None, :])
    dstl = dstl.reshape(4 * ROWS, CW)
    zero_deg = jnp.zeros((DPS, D), _f32)
    zero_q = jnp.zeros((SPS, D), _f32)

    dop8, dip8 = _sc_deg(e8, i8s, ids_, i8d, idd, zero_deg)
    dop = dop8.reshape(NC, NP, 16).sum(axis=0)[:N]
    dip = dip8.reshape(NC, NP, 16).sum(axis=0)[:N]
    xp = _tc_prep(x, dop)
    agg1 = _sc_feat(xp, src2d, dstl, zero_q)[:N]
    h1s = _tc_layer(agg1, dip, dop, W1, b1.reshape(1, D))
    agg2 = _sc_feat(h1s, src2d, dstl, zero_q)[:N]
    out = _tc_final(agg2, dip, x, W2, b2.reshape(1, D))
    return out


# speedup vs baseline: 1.8113x; 1.0488x over previous
import functools

import jax
import jax.numpy as jnp
from jax import lax
from jax.experimental import pallas as pl
from jax.experimental.pallas import tpu as pltpu
from jax.experimental.pallas import tpu_sc as plsc

N = 10000
E = 320000
D = 128

NC = 2
NS = 16

CW = 125
ROWS = E // CW
RPW = ROWS // NS
CPW = ROWS // (NC * NS)
NP = 10240
DS = NP // 8
DPS = DS // NS
QN = NP // 4
QP = QN + 8 * NS
SPS = QN // NS

_MESH = plsc.VectorSubcoreMesh(core_axis_name="c", subcore_axis_name="s")

_f32 = jnp.float32


@functools.partial(
    pl.kernel,
    out_type=(
        jax.ShapeDtypeStruct((NC * DS, D), _f32),
        jax.ShapeDtypeStruct((NC * DS, D), _f32),
    ),
    mesh=_MESH,
    scratch_types=[
        pltpu.VMEM((CPW, CW), jnp.int32),
        pltpu.VMEM((CPW, CW), jnp.int32),
        pltpu.VMEM((CW, D), _f32),
        pltpu.VMEM((DPS, D), _f32),
        pltpu.VMEM_SHARED((DS, D), _f32),
    ],
)
def _sc_deg(e8_hbm, i8s_hbm, ids_hbm, i8d_hbm, idd_hbm, zero_hbm,
            dop_hbm, dip_hbm,
            i8_v, id_v, buf, zb, acc):
    c = lax.axis_index("c")
    s = lax.axis_index("s")
    w = s * NC + c
    base = w * CPW
    for i8_hbm, id_hbm, o_hbm in ((i8s_hbm, ids_hbm, dop_hbm),
                                  (i8d_hbm, idd_hbm, dip_hbm)):
        pltpu.sync_copy(i8_hbm.at[pl.ds(base, CPW)], i8_v)
        pltpu.sync_copy(id_hbm.at[pl.ds(base, CPW)], id_v)
        pltpu.sync_copy(zero_hbm, zb)
        pltpu.sync_copy(zb, acc.at[pl.ds(s * DPS, DPS)])
        plsc.subcore_barrier()

        def body(j, _):
            pltpu.sync_copy(e8_hbm.at[i8_v.at[j]], buf)
            pltpu.sync_copy(buf, acc.at[id_v.at[j]], add=True)
            return 0

        lax.fori_loop(0, CPW, body, 0)
        plsc.subcore_barrier()
        pltpu.sync_copy(acc.at[pl.ds(s * DPS, DPS)], zb)
        pltpu.sync_copy(zb, o_hbm.at[pl.ds(c * DS + s * DPS, DPS)])


@functools.partial(
    pl.kernel,
    out_type=jax.ShapeDtypeStruct((4 * QN, D), _f32),
    mesh=_MESH,
    scratch_types=[
        pltpu.VMEM((RPW, CW), jnp.int32),
        pltpu.VMEM((RPW, CW), jnp.int32),
        pltpu.VMEM((CW, D), _f32),
        pltpu.VMEM((CW, D), _f32),
        pltpu.VMEM((SPS, D), _f32),
        pltpu.VMEM_SHARED((QP, D), _f32),
        pltpu.SemaphoreType.DMA,
        pltpu.SemaphoreType.DMA,
    ],
)
def _sc_feat(h_hbm, src_hbm, dstl_hbm, zero_hbm,
             out_hbm,
             src_idx, dst_idx, buf0, buf1, zb, acc, sem0, sem1):
    c = lax.axis_index("c")
    s = lax.axis_index("s")
    base = s * RPW

    pltpu.sync_copy(src_hbm.at[pl.ds(base, RPW)], src_idx)
    pltpu.sync_copy(zero_hbm, zb)
    bufs = (buf0, buf1)
    sems = (sem0, sem1)

    def gather(j, b):
        return pltpu.make_async_copy(
            h_hbm.at[src_idx.at[j]], bufs[b], sems[b])

    for p in range(2):
        q = 2 * p + c
        pltpu.sync_copy(dstl_hbm.at[pl.ds(q * ROWS + base, RPW)], dst_idx)
        pltpu.sync_copy(zb, acc.at[pl.ds(s * SPS, SPS)])
        pltpu.sync_copy(zb.at[pl.ds(0, 8)], acc.at[pl.ds(QN + 8 * s, 8)])
        plsc.subcore_barrier()

        gather(0, 0).start()
        gather(1, 1).start()

        def body(t, _):
            for b in range(2):
                j = 2 * t + b
                gather(j, b).wait()
                pltpu.sync_copy(bufs[b], acc.at[dst_idx.at[j]], add=True)
                gather(j + 2, b).start()
            return 0

        lax.fori_loop(0, RPW // 2 - 1, body, 0)
        for b in range(2):
            j = RPW - 2 + b
            gather(j, b).wait()
            pltpu.sync_copy(bufs[b], acc.at[dst_idx.at[j]], add=True)

        plsc.subcore_barrier()
        pltpu.sync_copy(acc.at[pl.ds(s * SPS, SPS)], zb)
        pltpu.sync_copy(zb, out_hbm.at[pl.ds(q * QN + s * SPS, SPS)])
        pltpu.sync_copy(zero_hbm, zb)


_R = 2000


def _deg_r(p_ref):
    return lax.rsqrt(jnp.maximum(p_ref[:, 0], 1.0))


def _prep_body(x_ref, dop_ref, o_ref):
    o_ref[...] = x_ref[...] * _deg_r(dop_ref)[:, None]


def _gcn_post(aggp_ref, dip_ref, w_ref, b_ref):
    agg = aggp_ref[...] * _deg_r(dip_ref)[:, None]
    z = jnp.dot(agg, w_ref[...], preferred_element_type=_f32) + b_ref[...]
    h = jnp.maximum(z, 0.0)
    nrm = jnp.sqrt(jnp.sum(h * h, axis=1, keepdims=True))
    return h / jnp.maximum(nrm, 1e-12)


def _layer_body(aggp_ref, dip_ref, dop_ref, w_ref, b_ref, o_ref):
    h = _gcn_post(aggp_ref, dip_ref, w_ref, b_ref)
    o_ref[...] = h * _deg_r(dop_ref)[:, None]


def _final_body(aggp_ref, dip_ref, x_ref, w_ref, b_ref, o_ref):
    h = _gcn_post(aggp_ref, dip_ref, w_ref, b_ref)
    y = jnp.maximum(h + x_ref[...], 0.0)
    nrm = jnp.sqrt(jnp.sum(y * y, axis=1, keepdims=True))
    o_ref[...] = y / jnp.maximum(nrm, 1e-12)


_spec_rows = pl.BlockSpec((_R, D), lambda i: (i, 0))
_spec_deg = pl.BlockSpec((_R, 16), lambda i: (i, 0))
_spec_w = pl.BlockSpec((D, D), lambda i: (0, 0))
_spec_b = pl.BlockSpec((1, D), lambda i: (0, 0))
_out_rows = jax.ShapeDtypeStruct((N, D), _f32)

_tc_prep = pl.pallas_call(
    _prep_body,
    grid=(N // _R,),
    in_specs=[_spec_rows, _spec_deg],
    out_specs=_spec_rows,
    out_shape=_out_rows,
)

_tc_layer = pl.pallas_call(
    _layer_body,
    grid=(N // _R,),
    in_specs=[_spec_rows, _spec_deg, _spec_deg, _spec_w, _spec_b],
    out_specs=_spec_rows,
    out_shape=_out_rows,
)

_tc_final = pl.pallas_call(
    _final_body,
    grid=(N // _R,),
    in_specs=[_spec_rows, _spec_deg, _spec_rows, _spec_w, _spec_b],
    out_specs=_spec_rows,
    out_shape=_out_rows,
)


def kernel(x, edge_index, W1, b1, W2, b2):
    src = edge_index[0]
    dst = edge_index[1]
    src2d = src.reshape(ROWS, CW)
    e8 = jnp.zeros((8, D), _f32).at[jnp.arange(8), jnp.arange(8) * 16].set(1.0)
    i8s = (src % 8).reshape(ROWS, CW)
    ids_ = (src // 8).reshape(ROWS, CW)
    i8d = (dst % 8).reshape(ROWS, CW)
    idd = (dst // 8).reshape(ROWS, CW)
    dump = QN + 8 * (jnp.arange(E, dtype=jnp.int32) // (RPW * CW))
    loc = dst[None, :] - (jnp.arange(4, dtype=jnp.int32) * QN)[:, None]
    dstl = jnp.where((loc >= 0) & (loc < QN), loc, dump[None, :])
    dstl = dstl.reshape(4 * ROWS, CW)
    zero_deg = jnp.zeros((DPS, D), _f32)
    zero_q = jnp.zeros((SPS, D), _f32)

    dop8, dip8 = _sc_deg(e8, i8s, ids_, i8d, idd, zero_deg)
    dop = dop8.reshape(NC, NP, 16).sum(axis=0)[:N]
    dip = dip8.reshape(NC, NP, 16).sum(axis=0)[:N]
    xp = _tc_prep(x, dop)
    agg1 = _sc_feat(xp, src2d, dstl, zero_q)[:N]
    h1s = _tc_layer(agg1, dip, dop, W1, b1.reshape(1, D))
    agg2 = _sc_feat(h1s, src2d, dstl, zero_q)[:N]
    out = _tc_final(agg2, dip, x, W2, b2.reshape(1, D))
    return out


# speedup vs baseline: 4.9865x; 2.7529x over previous
import functools

import jax
import jax.numpy as jnp
from jax import lax
from jax.experimental import pallas as pl
from jax.experimental.pallas import tpu as pltpu
from jax.experimental.pallas import tpu_sc as plsc

N = 10000
E = 320000
D = 128

NC = 2
NS = 16

CW = 125
ROWS = E // CW
RPW = ROWS // NS
CPW = ROWS // (NC * NS)
NP = 10240
DS = NP // 8
DPS = DS // NS
QN = NP // 4
QP = QN + 8 * NS
SPS = QN // NS

_MESH = plsc.VectorSubcoreMesh(core_axis_name="c", subcore_axis_name="s")

_f32 = jnp.float32


@functools.partial(
    pl.kernel,
    out_type=(
        jax.ShapeDtypeStruct((NC * DS, D), _f32),
        jax.ShapeDtypeStruct((NC * DS, D), _f32),
    ),
    mesh=_MESH,
    scratch_types=[
        pltpu.VMEM((CPW, CW), jnp.int32),
        pltpu.VMEM((CPW, CW), jnp.int32),
        pltpu.VMEM((CW, D), _f32),
        pltpu.VMEM((CW, D), _f32),
        pltpu.VMEM((DPS, D), _f32),
        pltpu.VMEM_SHARED((DS, D), _f32),
        pltpu.SemaphoreType.DMA,
        pltpu.SemaphoreType.DMA,
    ],
)
def _sc_deg(e8_hbm, i8s_hbm, ids_hbm, i8d_hbm, idd_hbm, zero_hbm,
            dop_hbm, dip_hbm,
            i8_v, id_v, buf0, buf1, zb, acc, sem0, sem1):
    c = lax.axis_index("c")
    s = lax.axis_index("s")
    w = s * NC + c
    base = w * CPW
    bufs = (buf0, buf1)
    sems = (sem0, sem1)

    def gather(i8_v2, j, b):
        return pltpu.make_async_copy(
            e8_hbm.at[i8_v2.at[j]], bufs[b], sems[b])

    for i8_hbm, id_hbm, o_hbm in ((i8s_hbm, ids_hbm, dop_hbm),
                                  (i8d_hbm, idd_hbm, dip_hbm)):
        pltpu.sync_copy(i8_hbm.at[pl.ds(base, CPW)], i8_v)
        pltpu.sync_copy(id_hbm.at[pl.ds(base, CPW)], id_v)
        pltpu.sync_copy(zero_hbm, zb)
        pltpu.sync_copy(zb, acc.at[pl.ds(s * DPS, DPS)])
        plsc.subcore_barrier()

        gather(i8_v, 0, 0).start()
        gather(i8_v, 1, 1).start()

        def body(t, _):
            for b in range(2):
                j = 2 * t + b
                gather(i8_v, j, b).wait()
                pltpu.sync_copy(bufs[b], acc.at[id_v.at[j]], add=True)
                gather(i8_v, j + 2, b).start()
            return 0

        lax.fori_loop(0, CPW // 2 - 1, body, 0)
        for b in range(2):
            j = CPW - 2 + b
            gather(i8_v, j, b).wait()
            pltpu.sync_copy(bufs[b], acc.at[id_v.at[j]], add=True)

        plsc.subcore_barrier()
        pltpu.sync_copy(acc.at[pl.ds(s * DPS, DPS)], zb)
        pltpu.sync_copy(zb, o_hbm.at[pl.ds(c * DS + s * DPS, DPS)])


@functools.partial(
    pl.kernel,
    out_type=jax.ShapeDtypeStruct((4 * QN, D), _f32),
    mesh=_MESH,
    scratch_types=[
        pltpu.VMEM((RPW, CW), jnp.int32),
        pltpu.VMEM((RPW, CW), jnp.int32),
        pltpu.VMEM((CW, D), _f32),
        pltpu.VMEM((CW, D), _f32),
        pltpu.VMEM((SPS, D), _f32),
        pltpu.VMEM_SHARED((QP, D), _f32),
        pltpu.SemaphoreType.DMA,
        pltpu.SemaphoreType.DMA,
    ],
)
def _sc_feat(h_hbm, src_hbm, dstl_hbm, zero_hbm,
             out_hbm,
             src_idx, dst_idx, buf0, buf1, zb, acc, sem0, sem1):
    c = lax.axis_index("c")
    s = lax.axis_index("s")
    base = s * RPW

    pltpu.sync_copy(src_hbm.at[pl.ds(base, RPW)], src_idx)
    pltpu.sync_copy(zero_hbm, zb)
    bufs = (buf0, buf1)
    sems = (sem0, sem1)

    def gather(j, b):
        return pltpu.make_async_copy(
            h_hbm.at[src_idx.at[j]], bufs[b], sems[b])

    for p in range(2):
        q = 2 * p + c
        pltpu.sync_copy(dstl_hbm.at[pl.ds(q * ROWS + base, RPW)], dst_idx)
        pltpu.sync_copy(zb, acc.at[pl.ds(s * SPS, SPS)])
        pltpu.sync_copy(zb.at[pl.ds(0, 8)], acc.at[pl.ds(QN + 8 * s, 8)])
        plsc.subcore_barrier()

        gather(0, 0).start()
        gather(1, 1).start()

        def body(t, _):
            for b in range(2):
                j = 2 * t + b
                gather(j, b).wait()
                pltpu.sync_copy(bufs[b], acc.at[dst_idx.at[j]], add=True)
                gather(j + 2, b).start()
            return 0

        lax.fori_loop(0, RPW // 2 - 1, body, 0)
        for b in range(2):
            j = RPW - 2 + b
            gather(j, b).wait()
            pltpu.sync_copy(bufs[b], acc.at[dst_idx.at[j]], add=True)

        plsc.subcore_barrier()
        pltpu.sync_copy(acc.at[pl.ds(s * SPS, SPS)], zb)
        pltpu.sync_copy(zb, out_hbm.at[pl.ds(q * QN + s * SPS, SPS)])
        pltpu.sync_copy(zero_hbm, zb)


_R = 2000


def _deg_r(p_ref):
    return lax.rsqrt(jnp.maximum(p_ref[:, 0], 1.0))


def _prep_body(x_ref, dop_ref, o_ref):
    o_ref[...] = x_ref[...] * _deg_r(dop_ref)[:, None]


def _gcn_post(aggp_ref, dip_ref, w_ref, b_ref):
    agg = aggp_ref[...] * _deg_r(dip_ref)[:, None]
    z = jnp.dot(agg, w_ref[...], preferred_element_type=_f32) + b_ref[...]
    h = jnp.maximum(z, 0.0)
    nrm = jnp.sqrt(jnp.sum(h * h, axis=1, keepdims=True))
    return h / jnp.maximum(nrm, 1e-12)


def _layer_body(aggp_ref, dip_ref, dop_ref, w_ref, b_ref, o_ref):
    h = _gcn_post(aggp_ref, dip_ref, w_ref, b_ref)
    o_ref[...] = h * _deg_r(dop_ref)[:, None]


def _final_body(aggp_ref, dip_ref, x_ref, w_ref, b_ref, o_ref):
    h = _gcn_post(aggp_ref, dip_ref, w_ref, b_ref)
    y = jnp.maximum(h + x_ref[...], 0.0)
    nrm = jnp.sqrt(jnp.sum(y * y, axis=1, keepdims=True))
    o_ref[...] = y / jnp.maximum(nrm, 1e-12)


_spec_rows = pl.BlockSpec((_R, D), lambda i: (i, 0))
_spec_deg = pl.BlockSpec((_R, 16), lambda i: (i, 0))
_spec_w = pl.BlockSpec((D, D), lambda i: (0, 0))
_spec_b = pl.BlockSpec((1, D), lambda i: (0, 0))
_out_rows = jax.ShapeDtypeStruct((N, D), _f32)

_tc_prep = pl.pallas_call(
    _prep_body,
    grid=(N // _R,),
    in_specs=[_spec_rows, _spec_deg],
    out_specs=_spec_rows,
    out_shape=_out_rows,
)

_tc_layer = pl.pallas_call(
    _layer_body,
    grid=(N // _R,),
    in_specs=[_spec_rows, _spec_deg, _spec_deg, _spec_w, _spec_b],
    out_specs=_spec_rows,
    out_shape=_out_rows,
)

_tc_final = pl.pallas_call(
    _final_body,
    grid=(N // _R,),
    in_specs=[_spec_rows, _spec_deg, _spec_rows, _spec_w, _spec_b],
    out_specs=_spec_rows,
    out_shape=_out_rows,
)


def kernel(x, edge_index, W1, b1, W2, b2):
    src = edge_index[0]
    dst = edge_index[1]
    src2d = src.reshape(ROWS, CW)
    e8 = jnp.zeros((8, D), _f32).at[jnp.arange(8), jnp.arange(8) * 16].set(1.0)
    e8 = jnp.tile(e8, (NC * NS, 1))
    woff = 8 * (jnp.arange(E, dtype=jnp.int32) // (CPW * CW))
    i8s = (src % 8 + woff).reshape(ROWS, CW)
    ids_ = (src // 8).reshape(ROWS, CW)
    i8d = (dst % 8 + woff).reshape(ROWS, CW)
    idd = (dst // 8).reshape(ROWS, CW)
    dump = QN + 8 * (jnp.arange(E, dtype=jnp.int32) // (RPW * CW))
    loc = dst[None, :] - (jnp.arange(4, dtype=jnp.int32) * QN)[:, None]
    dstl = jnp.where((loc >= 0) & (loc < QN), loc, dump[None, :])
    dstl = dstl.reshape(4 * ROWS, CW)
    zero_deg = jnp.zeros((DPS, D), _f32)
    zero_q = jnp.zeros((SPS, D), _f32)

    dop8, dip8 = _sc_deg(e8, i8s, ids_, i8d, idd, zero_deg)
    dop = dop8.reshape(NC, NP, 16).sum(axis=0)[:N]
    dip = dip8.reshape(NC, NP, 16).sum(axis=0)[:N]
    xp = _tc_prep(x, dop)
    agg1 = _sc_feat(xp, src2d, dstl, zero_q)[:N]
    h1s = _tc_layer(agg1, dip, dop, W1, b1.reshape(1, D))
    agg2 = _sc_feat(h1s, src2d, dstl, zero_q)[:N]
    out = _tc_final(agg2, dip, x, W2, b2.reshape(1, D))
    return out
